# baseline (speedup 1.0000x reference)
import functools

import jax
import jax.numpy as jnp
from jax import lax
from jax.experimental import pallas as pl
from jax.experimental.pallas import tpu as pltpu
from jax.experimental.pallas import tpu_sc as plsc

_NUM_NODES = 1000000
_DIM = 64
_BATCH = 16384

_NC = 2
_NS = 16
_NW = _NC * _NS
_BPW = _BATCH // _NW
_CHUNK = 128
_NCH = _BPW // _CHUNK
_LANES = 16
_PADDED = 2 * _DIM
_SPLIT = 499968
_TAILBASE = 2 * _SPLIT
_NTAIL = _NUM_NODES - _TAILBASE

_TCHUNK = 768
_TGRID = _SPLIT // _TCHUNK

_mesh = plsc.VectorSubcoreMesh(core_axis_name="c", subcore_axis_name="s")


def _transpose_body(a_ref, b_ref, o_ref):
    o_ref[:, 0:_DIM] = a_ref[...].T
    o_ref[:, _DIM:_PADDED] = b_ref[...].T


_transpose = pl.pallas_call(
    _transpose_body,
    grid=(_TGRID,),
    in_specs=[
        pl.BlockSpec((_DIM, _TCHUNK), lambda g: (0, g)),
        pl.BlockSpec((_DIM, _TCHUNK), lambda g: (0, g + _TGRID)),
    ],
    out_specs=pl.BlockSpec((_TCHUNK, _PADDED), lambda g: (g, 0)),
    out_shape=jax.ShapeDtypeStruct((_SPLIT, _PADDED), jnp.float32),
)


@functools.partial(
    pl.kernel,
    mesh=_mesh,
    out_type=jax.ShapeDtypeStruct((_DIM, _BATCH), jnp.float32),
    scratch_types=[
        pltpu.VMEM((_NCH, _CHUNK), jnp.int32),
        pltpu.VMEM((_NCH, _CHUNK), jnp.int32),
        pltpu.VMEM((_NCH, _CHUNK, _PADDED), jnp.float32),
        pltpu.VMEM((_DIM, _BPW), jnp.float32),
        pltpu.VMEM((_NTAIL, _DIM), jnp.float32),
        pltpu.SemaphoreType.DMA,
    ],
    compiler_params=pltpu.CompilerParams(needs_layout_passes=False),
)
def _mem_gather(n_id_hbm, memp_hbm, tail_hbm, memt_out,
                idx_v, pidx_v, prow_v, outt_v, tail_v, sem_m):
    wid = lax.axis_index("s") * _NC + lax.axis_index("c")
    base = wid * _BPW
    pltpu.sync_copy(n_id_hbm.at[pl.ds(wid * _NCH, _NCH)], idx_v)
    pltpu.sync_copy(tail_hbm, tail_v)
    for j in range(_NCH):
        for v in range(_CHUNK // _LANES):
            sl = pl.ds(v * _LANES, _LANES)
            n = idx_v[j, sl]
            p = n - jnp.where(n >= _SPLIT, _SPLIT, 0)
            pidx_v[j, sl] = jnp.where(n >= _TAILBASE, 0, p)
    row_copies = [
        pltpu.async_copy(memp_hbm.at[pidx_v.at[j]], prow_v.at[j], sem_m)
        for j in range(_NCH)
    ]
    for j in range(_NCH):
        row_copies[j].wait()
        def _select(g, carry, j=j):
            lanes = jax.lax.broadcasted_iota(jnp.int32, (_LANES,), 0)
            rvec = g * _LANES + lanes
            nvec = plsc.load_gather(idx_v.at[j], [rvec])
            half = jnp.where(nvec >= _SPLIT, _DIM, 0)
            is_tail = nvec >= _TAILBASE
            tn = jnp.where(is_tail, nvec - _TAILBASE, 0)
            for d in range(_DIM):
                vals = plsc.load_gather(prow_v.at[j], [rvec, half + d])
                tvals = plsc.load_gather(tail_v, [tn, tn * 0 + d])
                vals = jnp.where(is_tail, tvals, vals)
                outt_v[d, pl.ds(j * _CHUNK + g * _LANES, _LANES)] = vals
            return carry
        lax.fori_loop(0, _CHUNK // _LANES, _select, None)

    pltpu.sync_copy(outt_v, memt_out.at[:, pl.ds(base, _BPW)])


@functools.partial(
    pl.kernel,
    mesh=_mesh,
    out_type=jax.ShapeDtypeStruct((_BATCH,), jnp.float32),
    scratch_types=[
        pltpu.VMEM((_NCH, _CHUNK), jnp.int32),
        pltpu.VMEM((_NCH, _CHUNK), jnp.float32),
        pltpu.SemaphoreType.DMA,
    ],
)
def _lu_gather(n_id_hbm, lu_hbm, lu_out, idx_v, lu_v, sem_l):
    wid = lax.axis_index("s") * _NC + lax.axis_index("c")
    base = wid * _BPW
    pltpu.sync_copy(n_id_hbm.at[pl.ds(wid * _NCH, _NCH)], idx_v)
    lu_copies = [
        pltpu.async_copy(lu_hbm.at[idx_v.at[j]], lu_v.at[j], sem_l)
        for j in range(_NCH)
    ]
    for j in range(_NCH):
        lu_copies[j].wait()
        pltpu.sync_copy(lu_v.at[j], lu_out.at[pl.ds(base + j * _CHUNK, _CHUNK)])


def kernel(n_id, memory, last_update):
    n_id2 = n_id.astype(jnp.int32).reshape(_NW * _NCH, _CHUNK)
    lu_out = _lu_gather(n_id2, last_update)
    memp = _transpose(memory.T, memory.T)
    tail = memory[_TAILBASE:]
    memt_out = _mem_gather(n_id2, memp, tail)
    return (memt_out.T, lu_out)


# speedup vs baseline: 1.5788x; 1.5788x over previous
import functools

import jax
import jax.numpy as jnp
from jax import lax
from jax.experimental import pallas as pl
from jax.experimental.pallas import tpu as pltpu
from jax.experimental.pallas import tpu_sc as plsc

_NUM_NODES = 1000000
_DIM = 64
_BATCH = 16384

_NC = 2
_NS = 16
_NW = _NC * _NS
_BPW = _BATCH // _NW
_CHUNK = 128
_NCH = _BPW // _CHUNK
_LANES = 16
_PADDED = 2 * _DIM
_HALF = _NUM_NODES // 2

_mesh = plsc.VectorSubcoreMesh(core_axis_name="c", subcore_axis_name="s")


@functools.partial(
    pl.kernel,
    mesh=_mesh,
    out_type=jax.ShapeDtypeStruct((_DIM, _BATCH), jnp.float32),
    scratch_types=[
        pltpu.VMEM((_NCH, _CHUNK), jnp.int32),
        pltpu.VMEM((_NCH, _CHUNK), jnp.int32),
        pltpu.VMEM((_NCH, _CHUNK, _PADDED), jnp.float32),
        pltpu.VMEM((_DIM, _BPW), jnp.float32),
        pltpu.SemaphoreType.DMA,
    ],
    compiler_params=pltpu.CompilerParams(needs_layout_passes=False),
)
def _mem_gather(n_id_hbm, memp_hbm, memt_out,
                idx_v, pidx_v, prow_v, outt_v, sem_m):
    wid = lax.axis_index("s") * _NC + lax.axis_index("c")
    base = wid * _BPW
    pltpu.sync_copy(n_id_hbm.at[pl.ds(wid * _NCH, _NCH)], idx_v)
    for j in range(_NCH):
        for v in range(_CHUNK // _LANES):
            sl = pl.ds(v * _LANES, _LANES)
            n = idx_v[j, sl]
            pidx_v[j, sl] = n - jnp.where(n >= _HALF, _HALF, 0)
    row_copies = [
        pltpu.async_copy(memp_hbm.at[pidx_v.at[j]], prow_v.at[j], sem_m)
        for j in range(_NCH)
    ]
    for j in range(_NCH):
        row_copies[j].wait()
        def _select(g, carry, j=j):
            lanes = jax.lax.broadcasted_iota(jnp.int32, (_LANES,), 0)
            rvec = g * _LANES + lanes
            nvec = plsc.load_gather(idx_v.at[j], [rvec])
            half = jnp.where(nvec >= _HALF, _DIM, 0)
            for d in range(_DIM):
                vals = plsc.load_gather(prow_v.at[j], [rvec, half + d])
                outt_v[d, pl.ds(j * _CHUNK + g * _LANES, _LANES)] = vals
            return carry
        lax.fori_loop(0, _CHUNK // _LANES, _select, None)

    pltpu.sync_copy(outt_v, memt_out.at[:, pl.ds(base, _BPW)])


@functools.partial(
    pl.kernel,
    mesh=_mesh,
    out_type=jax.ShapeDtypeStruct((_BATCH,), jnp.float32),
    scratch_types=[
        pltpu.VMEM((_NCH, _CHUNK), jnp.int32),
        pltpu.VMEM((_NCH, _CHUNK), jnp.float32),
        pltpu.SemaphoreType.DMA,
    ],
)
def _lu_gather(n_id_hbm, lu_hbm, lu_out, idx_v, lu_v, sem_l):
    wid = lax.axis_index("s") * _NC + lax.axis_index("c")
    base = wid * _BPW
    pltpu.sync_copy(n_id_hbm.at[pl.ds(wid * _NCH, _NCH)], idx_v)
    lu_copies = [
        pltpu.async_copy(lu_hbm.at[idx_v.at[j]], lu_v.at[j], sem_l)
        for j in range(_NCH)
    ]
    for j in range(_NCH):
        lu_copies[j].wait()
        pltpu.sync_copy(lu_v.at[j], lu_out.at[pl.ds(base + j * _CHUNK, _CHUNK)])


def kernel(n_id, memory, last_update):
    n_id2 = n_id.astype(jnp.int32).reshape(_NW * _NCH, _CHUNK)
    lu_out = _lu_gather(n_id2, last_update)
    eye = jnp.eye(_DIM, dtype=jnp.float32)
    zero = jnp.zeros((_DIM, _DIM), jnp.float32)
    proj_l = jnp.concatenate([eye, zero], axis=1)
    proj_r = jnp.concatenate([zero, eye], axis=1)
    memp = (jax.lax.dot(memory[:_HALF], proj_l)
            + jax.lax.dot(memory[_HALF:], proj_r))
    memt_out = _mem_gather(n_id2, memp)
    return (memt_out.T, lu_out)


# speedup vs baseline: 2.1883x; 1.3861x over previous
import functools

import jax
import jax.numpy as jnp
from jax import lax
from jax.experimental import pallas as pl
from jax.experimental.pallas import tpu as pltpu
from jax.experimental.pallas import tpu_sc as plsc

_NUM_NODES = 1000000
_DIM = 64
_BATCH = 16384

_NC = 2
_NS = 16
_NW = _NC * _NS
_BPW = _BATCH // _NW
_CHUNK = 128
_NCH = _BPW // _CHUNK
_PADDED = 2 * _DIM

_mesh = plsc.VectorSubcoreMesh(core_axis_name="c", subcore_axis_name="s")


@functools.partial(
    pl.kernel,
    mesh=_mesh,
    out_type=jax.ShapeDtypeStruct((_BATCH, _PADDED), jnp.float32),
    scratch_types=[
        pltpu.VMEM((_NCH, _CHUNK), jnp.int32),
        pltpu.VMEM((_NCH, _CHUNK, _PADDED), jnp.float32),
        pltpu.SemaphoreType.DMA,
    ],
)
def _mem_gather(n_id_hbm, memp_hbm, mem_out, idx_v, row_v, sem_m):
    wid = lax.axis_index("s") * _NC + lax.axis_index("c")
    base = wid * _BPW
    pltpu.sync_copy(n_id_hbm.at[pl.ds(wid * _NCH, _NCH)], idx_v)
    row_copies = [
        pltpu.async_copy(memp_hbm.at[idx_v.at[j]], row_v.at[j], sem_m)
        for j in range(_NCH)
    ]
    for j in range(_NCH):
        row_copies[j].wait()
        pltpu.sync_copy(
            row_v.at[j], mem_out.at[pl.ds(base + j * _CHUNK, _CHUNK)])


@functools.partial(
    pl.kernel,
    mesh=_mesh,
    out_type=jax.ShapeDtypeStruct((_BATCH,), jnp.float32),
    scratch_types=[
        pltpu.VMEM((_NCH, _CHUNK), jnp.int32),
        pltpu.VMEM((_NCH, _CHUNK), jnp.float32),
        pltpu.SemaphoreType.DMA,
    ],
)
def _lu_gather(n_id_hbm, lu_hbm, lu_out, idx_v, lu_v, sem_l):
    wid = lax.axis_index("s") * _NC + lax.axis_index("c")
    base = wid * _BPW
    pltpu.sync_copy(n_id_hbm.at[pl.ds(wid * _NCH, _NCH)], idx_v)
    lu_copies = [
        pltpu.async_copy(lu_hbm.at[idx_v.at[j]], lu_v.at[j], sem_l)
        for j in range(_NCH)
    ]
    for j in range(_NCH):
        lu_copies[j].wait()
        pltpu.sync_copy(lu_v.at[j], lu_out.at[pl.ds(base + j * _CHUNK, _CHUNK)])


def kernel(n_id, memory, last_update):
    n_id2 = n_id.astype(jnp.int32).reshape(_NW * _NCH, _CHUNK)
    lu_out = _lu_gather(n_id2, last_update)
    proj = jnp.concatenate(
        [jnp.eye(_DIM, dtype=jnp.float32),
         jnp.zeros((_DIM, _DIM), jnp.float32)], axis=1)
    memp = jax.lax.dot(memory, proj)
    mem_out = _mem_gather(n_id2, memp)
    return (mem_out[:, :_DIM], lu_out)
